# baseline (speedup 1.0000x reference)
import functools

import jax
import jax.numpy as jnp
from jax import lax
from jax.experimental import pallas as pl
from jax.experimental.pallas import tpu as pltpu
from jax.experimental.pallas import tpu_sc as plsc

_N = 10000
_E = 320000
_D = 128
_NC = 2
_NS = 16
_NW = _NC * _NS
_EPW = _E // _NW
_CH = 80
_NCHUNK = _EPW // _CH

_mesh = plsc.VectorSubcoreMesh(core_axis_name="c", subcore_axis_name="s")


def _zero16():
    return jnp.zeros((16,), jnp.float32)


def _zero_rows(buf):
    def _z(i, _):
        for j in range(_D // 16):
            buf[i, pl.ds(j * 16, 16)] = _zero16()
        return 0
    lax.fori_loop(0, _CH, _z, 0)


def _zero_acc_slice(zsrc, acc, s):
    for k in range(640 // _CH):
        pltpu.sync_copy(zsrc, acc.at[pl.ds(s * 624 + k * _CH, _CH)])


def _copy_out(acc, out, c, s):
    pltpu.sync_copy(acc.at[pl.ds(s * 624, 640)],
                    out.at[c, pl.ds(s * 624, 640)])


def _deg_body(edata1, degpart, b0, b1, b2, b3, q0, q1, q2, q3, v0, v1, acc,
              lsem, s0, s1):
    c = lax.axis_index("c")
    s = lax.axis_index("s")
    wid = c * _NS + s
    base = wid * _NCHUNK * 3 * _CH

    dbuf = (b0, b1, b2, b3)
    sidx = (q0, q1, q2, q3)
    vals = (v0, v1)
    ssem = (s0, s1)

    _zero_rows(v0)
    _zero_rows(v1)
    _zero_acc_slice(v0, acc, s)
    plsc.subcore_barrier()

    def _dload(ci, x):
        pltpu.async_copy(edata1.at[pl.ds(base + ci * 3 * _CH + _CH, 2 * _CH)],
                         dbuf[x], lsem)

    def _dwait(ci, x):
        pltpu.make_async_copy(
            edata1.at[pl.ds(base + ci * 3 * _CH + _CH, 2 * _CH)],
            dbuf[x], lsem).wait()

    def _fill_snap(ci, x, y):
        val = vals[y]

        def _f(k, _):
            w16 = lax.bitcast_convert_type(
                dbuf[x][pl.ds(_CH + k * 16, 16)], jnp.float32)
            for l in range(16):
                val[k * 16 + l, pl.ds(0, 16)] = jnp.full((16,), w16[l],
                                                         jnp.float32)
            return 0
        lax.fori_loop(0, _CH // 16, _f, 0)
        for k in range(_CH // 16):
            sidx[x][pl.ds(k * 16, 16)] = dbuf[x][pl.ds(k * 16, 16)]

    def _scatter(ci, x, y):
        pltpu.async_copy(vals[y], acc.at[sidx[x]], ssem[y], add=True)

    def _swait(ci, x, y):
        pltpu.make_async_copy(vals[y], acc.at[sidx[x]], ssem[y]).wait()

    def _step(ci, x, y):
        _dwait(ci, x)

        @pl.when(ci >= 2)
        def _():
            _swait(ci - 2, (x + 2) % 4, y)

        @pl.when(ci + 2 < _NCHUNK)
        def _():
            _dload(ci + 2, (x + 2) % 4)

        _fill_snap(ci, x, y)
        _scatter(ci, x, y)

    _dload(0, 0)
    _dload(1, 1)

    def _quad(k, _):
        for x in range(4):
            _step(4 * k + x, x, x % 2)
        return 0
    lax.fori_loop(0, _NCHUNK // 4, _quad, 0)
    _step(_NCHUNK - 1, (_NCHUNK - 1) % 4, (_NCHUNK - 1) % 2)
    _swait(_NCHUNK - 2, (_NCHUNK - 2) % 4, (_NCHUNK - 2) % 2)
    _swait(_NCHUNK - 1, (_NCHUNK - 1) % 4, (_NCHUNK - 1) % 2)

    plsc.subcore_barrier()
    _copy_out(acc, degpart, c, s)


def _deg_call(edata1):
    k = pl.kernel(
        _deg_body,
        out_type=jax.ShapeDtypeStruct((_NC, _N, _D), jnp.float32),
        mesh=_mesh,
        scratch_types=(
            [pltpu.VMEM((2 * _CH,), jnp.int32)] * 4
            + [pltpu.VMEM((_CH,), jnp.int32)] * 4
            + [pltpu.VMEM((_CH, _D), jnp.float32)] * 2
            + [pltpu.VMEM_SHARED((_N, _D), jnp.float32)]
            + [pltpu.SemaphoreType.DMA] * 3
        ),
    )
    return k(edata1)


def _prop_body(nj, edata1, g, spart,
               b0, b1, b2, b3, q0, q1, q2, q3, r0, r1, r2, r3, acc,
               lsem, g0, g1, g2, g3, s0, s1, s2, s3):
    c = lax.axis_index("c")
    s = lax.axis_index("s")
    wid = c * _NS + s
    base = wid * _NCHUNK * 3 * _CH

    ebuf = (b0, b1, b2, b3)
    sidx = (q0, q1, q2, q3)
    rows = (r0, r1, r2, r3)
    gsem = (g0, g1, g2, g3)
    ssem = (s0, s1, s2, s3)

    _zero_rows(r0)
    _zero_acc_slice(r0, acc, s)
    plsc.subcore_barrier()

    def _load(ci, x):
        pltpu.async_copy(edata1.at[pl.ds(base + ci * 3 * _CH, 3 * _CH)],
                         ebuf[x], lsem)

    def _lwait(ci, x):
        pltpu.make_async_copy(edata1.at[pl.ds(base + ci * 3 * _CH, 3 * _CH)],
                              ebuf[x], lsem).wait()

    def _gather(ci, x):
        pltpu.async_copy(g.at[ebuf[x].at[pl.ds(0, _CH)]], rows[x], gsem[x])

    def _gwait(ci, x):
        pltpu.make_async_copy(g.at[ebuf[x].at[pl.ds(0, _CH)]], rows[x],
                              gsem[x]).wait()

    def _scatter(ci, x):
        pltpu.async_copy(rows[x], acc.at[sidx[x]], ssem[x], add=True)

    def _swait(ci, x):
        pltpu.make_async_copy(rows[x], acc.at[sidx[x]], ssem[x]).wait()

    def _scale_snap(ci, x):
        buf = rows[x]

        def _sc(k, _):
            w16 = lax.bitcast_convert_type(
                ebuf[x][pl.ds(2 * _CH + k * 16, 16)], jnp.float32)
            for l in range(16):
                w = w16[l]
                i = k * 16 + l
                for j in range(nj):
                    buf[i, pl.ds(j * 16, 16)] = buf[i, pl.ds(j * 16, 16)] * w
            return 0
        lax.fori_loop(0, _CH // 16, _sc, 0)
        for k in range(_CH // 16):
            sidx[x][pl.ds(k * 16, 16)] = ebuf[x][pl.ds(_CH + k * 16, 16)]

    def _step(ci, x):
        @pl.when(ci + 3 < _NCHUNK)
        def _():
            _load(ci + 3, (x + 3) % 4)

        @pl.when(ci + 2 < _NCHUNK)
        def _():
            _lwait(ci + 2, (x + 2) % 4)

            @pl.when(ci >= 2)
            def _():
                _swait(ci - 2, (x + 2) % 4)
            _gather(ci + 2, (x + 2) % 4)

        _gwait(ci, x)
        _scale_snap(ci, x)
        _scatter(ci, x)

    _load(0, 0)
    _load(1, 1)
    _load(2, 2)
    _lwait(0, 0)
    _gather(0, 0)
    _lwait(1, 1)
    _gather(1, 1)

    def _quad(k, _):
        for x in range(4):
            _step(4 * k + x, x)
        return 0
    lax.fori_loop(0, _NCHUNK // 4, _quad, 0)
    _step(_NCHUNK - 1, (_NCHUNK - 1) % 4)
    for ci in range(_NCHUNK - 4, _NCHUNK):
        _swait(ci, ci % 4)

    plsc.subcore_barrier()
    _copy_out(acc, spart, c, s)


def _prop_call(edata1, g, d_valid):
    nj = d_valid // 16
    k = pl.kernel(
        functools.partial(_prop_body, nj),
        out_type=jax.ShapeDtypeStruct((_NC, _N, _D), jnp.float32),
        mesh=_mesh,
        scratch_types=(
            [pltpu.VMEM((3 * _CH,), jnp.int32)] * 4
            + [pltpu.VMEM((_CH,), jnp.int32)] * 4
            + [pltpu.VMEM((_CH, _D), jnp.float32)] * 4
            + [pltpu.VMEM_SHARED((_N, _D), jnp.float32)]
            + [pltpu.SemaphoreType.DMA] * 9
        ),
    )
    return k(edata1, g)


_BLK = 1000


def _mm(a, w):
    return jnp.dot(a, w, preferred_element_type=jnp.float32,
                   precision=lax.Precision.HIGHEST)


def _tc_lin_body(x_r, w_r, out_r):
    out_r[...] = _mm(x_r[...], w_r[...])


def _tc_lin(x, W1p):
    return pl.pallas_call(
        _tc_lin_body,
        grid=(_N // _BLK,),
        in_specs=[
            pl.BlockSpec((_BLK, _D), lambda i: (i, 0)),
            pl.BlockSpec((_D, _D), lambda i: (0, 0)),
        ],
        out_specs=pl.BlockSpec((_BLK, _D), lambda i: (i, 0)),
        out_shape=jax.ShapeDtypeStruct((_N, _D), jnp.float32),
    )(x, W1p)


def _tc_gd_body(dp_r, lin_r, g_r, dv_r):
    deg = dp_r[0, :, 0] + dp_r[1, :, 0] + 1.0
    dinv = jnp.where(deg > 0, lax.rsqrt(deg), 0.0)
    g_r[...] = dinv[:, None] * lin_r[...]
    dv_r[...] = dinv[:, None]


def _tc_gd(degpart, lin):
    return pl.pallas_call(
        _tc_gd_body,
        grid=(_N // _BLK,),
        in_specs=[
            pl.BlockSpec((_NC, _BLK, _D), lambda i: (0, i, 0)),
            pl.BlockSpec((_BLK, _D), lambda i: (i, 0)),
        ],
        out_specs=(pl.BlockSpec((_BLK, _D), lambda i: (i, 0)),
                   pl.BlockSpec((_BLK, 1), lambda i: (i, 0))),
        out_shape=(jax.ShapeDtypeStruct((_N, _D), jnp.float32),
                   jax.ShapeDtypeStruct((_N, 1), jnp.float32)),
    )(degpart, lin)


def _tc_mid_body(final, s_r, g_r, dv_r, b_r, w_r, bd_r, out_r):
    dinv = dv_r[...][:, 0]
    stot = s_r[0] + s_r[1] + g_r[...]
    h = jax.nn.relu(dinv[:, None] * stot + b_r[...][None, :])
    z = _mm(h, w_r[...])
    if final:
        out_r[...] = z + bd_r[...][None, :]
    else:
        out_r[...] = dinv[:, None] * z


def _tc_mid(S, g, dinvcol, bp, Wp, bd, final):
    return pl.pallas_call(
        functools.partial(_tc_mid_body, final),
        grid=(_N // _BLK,),
        in_specs=[
            pl.BlockSpec((_NC, _BLK, _D), lambda i: (0, i, 0)),
            pl.BlockSpec((_BLK, _D), lambda i: (i, 0)),
            pl.BlockSpec((_BLK, 1), lambda i: (i, 0)),
            pl.BlockSpec((_D,), lambda i: (0,)),
            pl.BlockSpec((_D, _D), lambda i: (0, 0)),
            pl.BlockSpec((_D,), lambda i: (0,)),
        ],
        out_specs=pl.BlockSpec((_BLK, _D), lambda i: (i, 0)),
        out_shape=jax.ShapeDtypeStruct((_N, _D), jnp.float32),
    )(S, g, dinvcol, bp, Wp, bd)


def _pad2(w):
    return jnp.zeros((_D, _D), jnp.float32).at[:w.shape[0], :w.shape[1]].set(w)


def _pad1(b):
    return jnp.zeros((_D,), jnp.float32).at[:b.shape[0]].set(b)


def kernel(x, edgeIndex, edgeWeight, W1, b1, W2, b2, W3, b3, W4, b4, Wd, bd):
    src3 = edgeIndex[0].reshape(_NW, _NCHUNK, _CH)
    dst3 = edgeIndex[1].reshape(_NW, _NCHUNK, _CH)
    ewb3 = lax.bitcast_convert_type(edgeWeight, jnp.int32).reshape(
        _NW, _NCHUNK, _CH)
    edata1 = jnp.stack([src3, dst3, ewb3], axis=2).reshape(-1)

    zb = jnp.zeros((_D,), jnp.float32)
    degpart = _deg_call(edata1)
    lin = _tc_lin(x, _pad2(W1))
    g, dinvcol = _tc_gd(degpart, lin)
    for d_in, b, W in ((16, b1, W2), (32, b2, W3), (64, b3, W4)):
        S = _prop_call(edata1, g, d_in)
        g = _tc_mid(S, g, dinvcol, _pad1(b), _pad2(W), zb, final=False)
    S = _prop_call(edata1, g, 128)
    return _tc_mid(S, g, dinvcol, _pad1(b4), Wd, bd, final=True)


# speedup vs baseline: 1.0489x; 1.0489x over previous
import functools

import jax
import jax.numpy as jnp
from jax import lax
from jax.experimental import pallas as pl
from jax.experimental.pallas import tpu as pltpu
from jax.experimental.pallas import tpu_sc as plsc

_N = 10000
_E = 320000
_D = 128
_NC = 2
_NS = 16
_NW = _NC * _NS
_EPW = _E // _NW
_CH = 80
_NCHUNK = _EPW // _CH

_mesh = plsc.VectorSubcoreMesh(core_axis_name="c", subcore_axis_name="s")


def _zero16():
    return jnp.zeros((16,), jnp.float32)


def _zero_rows(buf):
    def _z(i, _):
        for j in range(_D // 16):
            buf[i, pl.ds(j * 16, 16)] = _zero16()
        return 0
    lax.fori_loop(0, _CH, _z, 0)


def _zero_acc_slice(zsrc, acc, s):
    for k in range(640 // _CH):
        pltpu.sync_copy(zsrc, acc.at[pl.ds(s * 624 + k * _CH, _CH)])


def _copy_out(acc, out, c, s):
    pltpu.sync_copy(acc.at[pl.ds(s * 624, 640)],
                    out.at[c, pl.ds(s * 624, 640)])


def _deg_body(dst_e, ew, degpart, b0, b1, b2, b3, w0, w1, w2, w3,
              q0, q1, q2, q3, v0, v1, acc, lsem, s0, s1):
    c = lax.axis_index("c")
    s = lax.axis_index("s")
    wid = c * _NS + s
    base = wid * _EPW

    wbuf = (w0, w1, w2, w3)

    dbuf = (b0, b1, b2, b3)
    sidx = (q0, q1, q2, q3)
    vals = (v0, v1)
    ssem = (s0, s1)

    _zero_rows(v0)
    _zero_rows(v1)
    _zero_acc_slice(v0, acc, s)
    plsc.subcore_barrier()

    def _dload(ci, x):
        off = base + ci * _CH
        pltpu.async_copy(dst_e.at[pl.ds(off, _CH)], dbuf[x], lsem)
        pltpu.async_copy(ew.at[pl.ds(off, _CH)], wbuf[x], lsem)

    def _dwait(ci, x):
        off = base + ci * _CH
        pltpu.make_async_copy(dst_e.at[pl.ds(off, _CH)], dbuf[x], lsem).wait()
        pltpu.make_async_copy(ew.at[pl.ds(off, _CH)], wbuf[x], lsem).wait()

    def _fill_snap(ci, x, y):
        val = vals[y]

        def _f(k, _):
            w16 = wbuf[x][pl.ds(k * 16, 16)]
            for l in range(16):
                val[k * 16 + l, pl.ds(0, 16)] = jnp.full((16,), w16[l],
                                                         jnp.float32)
            return 0
        lax.fori_loop(0, _CH // 16, _f, 0)
        for k in range(_CH // 16):
            sidx[x][pl.ds(k * 16, 16)] = dbuf[x][pl.ds(k * 16, 16)]

    def _scatter(ci, x, y):
        pltpu.async_copy(vals[y], acc.at[sidx[x]], ssem[y], add=True)

    def _swait(ci, x, y):
        pltpu.make_async_copy(vals[y], acc.at[sidx[x]], ssem[y]).wait()

    def _step(ci, x, y):
        _dwait(ci, x)

        @pl.when(ci >= 2)
        def _():
            _swait(ci - 2, (x + 2) % 4, y)

        @pl.when(ci + 2 < _NCHUNK)
        def _():
            _dload(ci + 2, (x + 2) % 4)

        _fill_snap(ci, x, y)
        _scatter(ci, x, y)

    _dload(0, 0)
    _dload(1, 1)

    def _quad(k, _):
        for x in range(4):
            _step(4 * k + x, x, x % 2)
        return 0
    lax.fori_loop(0, _NCHUNK // 4, _quad, 0)
    _step(_NCHUNK - 1, (_NCHUNK - 1) % 4, (_NCHUNK - 1) % 2)
    _swait(_NCHUNK - 2, (_NCHUNK - 2) % 4, (_NCHUNK - 2) % 2)
    _swait(_NCHUNK - 1, (_NCHUNK - 1) % 4, (_NCHUNK - 1) % 2)

    plsc.subcore_barrier()
    _copy_out(acc, degpart, c, s)


def _deg_call(dst_e, ew):
    k = pl.kernel(
        _deg_body,
        out_type=jax.ShapeDtypeStruct((_NC, _N, _D), jnp.float32),
        mesh=_mesh,
        scratch_types=(
            [pltpu.VMEM((_CH,), jnp.int32)] * 4
            + [pltpu.VMEM((_CH,), jnp.float32)] * 4
            + [pltpu.VMEM((_CH,), jnp.int32)] * 4
            + [pltpu.VMEM((_CH, _D), jnp.float32)] * 2
            + [pltpu.VMEM_SHARED((_N, _D), jnp.float32)]
            + [pltpu.SemaphoreType.DMA] * 3
        ),
    )
    return k(dst_e, ew)


def _prop_body(nj, src_e, dst_e, ew, g, spart,
               a0, a1, a2, a3, b0, b1, b2, b3, e0, e1, e2, e3,
               q0, q1, q2, q3, r0, r1, r2, r3, acc,
               lsem, g0, g1, g2, g3, s0, s1, s2, s3):
    c = lax.axis_index("c")
    s = lax.axis_index("s")
    wid = c * _NS + s
    base = wid * _EPW

    sch = (a0, a1, a2, a3)
    dch = (b0, b1, b2, b3)
    ech = (e0, e1, e2, e3)
    sidx = (q0, q1, q2, q3)
    rows = (r0, r1, r2, r3)
    gsem = (g0, g1, g2, g3)
    ssem = (s0, s1, s2, s3)

    _zero_rows(r0)
    _zero_acc_slice(r0, acc, s)
    plsc.subcore_barrier()

    def _load(ci, x):
        off = base + ci * _CH
        pltpu.async_copy(src_e.at[pl.ds(off, _CH)], sch[x], lsem)
        pltpu.async_copy(dst_e.at[pl.ds(off, _CH)], dch[x], lsem)
        pltpu.async_copy(ew.at[pl.ds(off, _CH)], ech[x], lsem)

    def _lwait(ci, x):
        off = base + ci * _CH
        pltpu.make_async_copy(src_e.at[pl.ds(off, _CH)], sch[x], lsem).wait()
        pltpu.make_async_copy(dst_e.at[pl.ds(off, _CH)], dch[x], lsem).wait()
        pltpu.make_async_copy(ew.at[pl.ds(off, _CH)], ech[x], lsem).wait()

    def _gather(ci, x):
        pltpu.async_copy(g.at[sch[x]], rows[x], gsem[x])

    def _gwait(ci, x):
        pltpu.make_async_copy(g.at[sch[x]], rows[x], gsem[x]).wait()

    def _scatter(ci, x):
        pltpu.async_copy(rows[x], acc.at[sidx[x]], ssem[x], add=True)

    def _swait(ci, x):
        pltpu.make_async_copy(rows[x], acc.at[sidx[x]], ssem[x]).wait()

    def _scale_snap(ci, x):
        buf = rows[x]

        def _sc(k, _):
            w16 = ech[x][pl.ds(k * 16, 16)]
            for l in range(16):
                w = w16[l]
                i = k * 16 + l
                for j in range(nj):
                    buf[i, pl.ds(j * 16, 16)] = buf[i, pl.ds(j * 16, 16)] * w
            return 0
        lax.fori_loop(0, _CH // 16, _sc, 0)
        for k in range(_CH // 16):
            sidx[x][pl.ds(k * 16, 16)] = dch[x][pl.ds(k * 16, 16)]

    def _step(ci, x):
        @pl.when(ci + 3 < _NCHUNK)
        def _():
            _load(ci + 3, (x + 3) % 4)

        @pl.when(ci + 2 < _NCHUNK)
        def _():
            _lwait(ci + 2, (x + 2) % 4)

            @pl.when(ci >= 2)
            def _():
                _swait(ci - 2, (x + 2) % 4)
            _gather(ci + 2, (x + 2) % 4)

        _gwait(ci, x)
        _scale_snap(ci, x)
        _scatter(ci, x)

    _load(0, 0)
    _load(1, 1)
    _load(2, 2)
    _lwait(0, 0)
    _gather(0, 0)
    _lwait(1, 1)
    _gather(1, 1)

    def _quad(k, _):
        for x in range(4):
            _step(4 * k + x, x)
        return 0
    lax.fori_loop(0, _NCHUNK // 4, _quad, 0)
    _step(_NCHUNK - 1, (_NCHUNK - 1) % 4)
    for ci in range(_NCHUNK - 4, _NCHUNK):
        _swait(ci, ci % 4)

    plsc.subcore_barrier()
    _copy_out(acc, spart, c, s)


def _prop_call(src_e, dst_e, ew, g, d_valid):
    nj = d_valid // 16
    k = pl.kernel(
        functools.partial(_prop_body, nj),
        out_type=jax.ShapeDtypeStruct((_NC, _N, _D), jnp.float32),
        mesh=_mesh,
        scratch_types=(
            [pltpu.VMEM((_CH,), jnp.int32)] * 8
            + [pltpu.VMEM((_CH,), jnp.float32)] * 4
            + [pltpu.VMEM((_CH,), jnp.int32)] * 4
            + [pltpu.VMEM((_CH, _D), jnp.float32)] * 4
            + [pltpu.VMEM_SHARED((_N, _D), jnp.float32)]
            + [pltpu.SemaphoreType.DMA] * 9
        ),
    )
    return k(src_e, dst_e, ew, g)


_BLK = 1000


def _mm(a, w):
    return jnp.dot(a, w, preferred_element_type=jnp.float32,
                   precision=lax.Precision.HIGHEST)


def _tc_lin_body(x_r, w_r, out_r):
    out_r[...] = _mm(x_r[...], w_r[...])


def _tc_lin(x, W1p):
    return pl.pallas_call(
        _tc_lin_body,
        grid=(_N // _BLK,),
        in_specs=[
            pl.BlockSpec((_BLK, _D), lambda i: (i, 0)),
            pl.BlockSpec((_D, _D), lambda i: (0, 0)),
        ],
        out_specs=pl.BlockSpec((_BLK, _D), lambda i: (i, 0)),
        out_shape=jax.ShapeDtypeStruct((_N, _D), jnp.float32),
    )(x, W1p)


def _tc_gd_body(dp_r, lin_r, g_r, dv_r):
    deg = dp_r[0, :, 0] + dp_r[1, :, 0] + 1.0
    dinv = jnp.where(deg > 0, lax.rsqrt(deg), 0.0)
    g_r[...] = dinv[:, None] * lin_r[...]
    dv_r[...] = dinv[:, None]


def _tc_gd(degpart, lin):
    return pl.pallas_call(
        _tc_gd_body,
        grid=(_N // _BLK,),
        in_specs=[
            pl.BlockSpec((_NC, _BLK, _D), lambda i: (0, i, 0)),
            pl.BlockSpec((_BLK, _D), lambda i: (i, 0)),
        ],
        out_specs=(pl.BlockSpec((_BLK, _D), lambda i: (i, 0)),
                   pl.BlockSpec((_BLK, 1), lambda i: (i, 0))),
        out_shape=(jax.ShapeDtypeStruct((_N, _D), jnp.float32),
                   jax.ShapeDtypeStruct((_N, 1), jnp.float32)),
    )(degpart, lin)


def _tc_mid_body(final, s_r, g_r, dv_r, b_r, w_r, bd_r, out_r):
    dinv = dv_r[...][:, 0]
    stot = s_r[0] + s_r[1] + g_r[...]
    h = jax.nn.relu(dinv[:, None] * stot + b_r[...][None, :])
    z = _mm(h, w_r[...])
    if final:
        out_r[...] = z + bd_r[...][None, :]
    else:
        out_r[...] = dinv[:, None] * z


def _tc_mid(S, g, dinvcol, bp, Wp, bd, final):
    return pl.pallas_call(
        functools.partial(_tc_mid_body, final),
        grid=(_N // _BLK,),
        in_specs=[
            pl.BlockSpec((_NC, _BLK, _D), lambda i: (0, i, 0)),
            pl.BlockSpec((_BLK, _D), lambda i: (i, 0)),
            pl.BlockSpec((_BLK, 1), lambda i: (i, 0)),
            pl.BlockSpec((_D,), lambda i: (0,)),
            pl.BlockSpec((_D, _D), lambda i: (0, 0)),
            pl.BlockSpec((_D,), lambda i: (0,)),
        ],
        out_specs=pl.BlockSpec((_BLK, _D), lambda i: (i, 0)),
        out_shape=jax.ShapeDtypeStruct((_N, _D), jnp.float32),
    )(S, g, dinvcol, bp, Wp, bd)


def _pad2(w):
    return jnp.zeros((_D, _D), jnp.float32).at[:w.shape[0], :w.shape[1]].set(w)


def _pad1(b):
    return jnp.zeros((_D,), jnp.float32).at[:b.shape[0]].set(b)


def kernel(x, edgeIndex, edgeWeight, W1, b1, W2, b2, W3, b3, W4, b4, Wd, bd):
    src_e = edgeIndex[0]
    dst_e = edgeIndex[1]
    zb = jnp.zeros((_D,), jnp.float32)
    degpart = _deg_call(dst_e, edgeWeight)
    lin = _tc_lin(x, _pad2(W1))
    g, dinvcol = _tc_gd(degpart, lin)
    for d_in, b, W in ((16, b1, W2), (32, b2, W3), (64, b3, W4)):
        S = _prop_call(src_e, dst_e, edgeWeight, g, d_in)
        g = _tc_mid(S, g, dinvcol, _pad1(b), _pad2(W), zb, final=False)
    S = _prop_call(src_e, dst_e, edgeWeight, g, 128)
    return _tc_mid(S, g, dinvcol, _pad1(b4), Wd, bd, final=True)


# speedup vs baseline: 1.0538x; 1.0046x over previous
import functools

import jax
import jax.numpy as jnp
from jax import lax
from jax.experimental import pallas as pl
from jax.experimental.pallas import tpu as pltpu
from jax.experimental.pallas import tpu_sc as plsc

_N = 10000
_E = 320000
_D = 128
_NC = 2
_NS = 16
_NW = _NC * _NS
_EPW = _E // _NW
_CH = 80
_NCHUNK = _EPW // _CH

_mesh = plsc.VectorSubcoreMesh(core_axis_name="c", subcore_axis_name="s")


def _zero16():
    return jnp.zeros((16,), jnp.float32)


def _zero_rows(buf):
    def _z(i, _):
        for j in range(_D // 16):
            buf[i, pl.ds(j * 16, 16)] = _zero16()
        return 0
    lax.fori_loop(0, _CH, _z, 0)


def _zero_acc_slice(zsrc, acc, s, zsem):
    for k in range(640 // _CH):
        pltpu.async_copy(zsrc, acc.at[pl.ds(s * 624 + k * _CH, _CH)], zsem)
    for k in range(640 // _CH):
        pltpu.make_async_copy(
            zsrc, acc.at[pl.ds(s * 624 + k * _CH, _CH)], zsem).wait()


def _copy_out(acc, out, c, s):
    pltpu.sync_copy(acc.at[pl.ds(s * 624, 640)],
                    out.at[c, pl.ds(s * 624, 640)])


def _deg_body(dst_e, ew, degpart, b0, b1, b2, b3, w0, w1, w2, w3,
              q0, q1, q2, q3, v0, v1, acc, lsem, s0, s1):
    c = lax.axis_index("c")
    s = lax.axis_index("s")
    wid = c * _NS + s
    base = wid * _EPW

    wbuf = (w0, w1, w2, w3)

    dbuf = (b0, b1, b2, b3)
    sidx = (q0, q1, q2, q3)
    vals = (v0, v1)
    ssem = (s0, s1)

    _zero_rows(v0)
    _zero_rows(v1)
    _zero_acc_slice(v0, acc, s, lsem)
    plsc.subcore_barrier()

    def _dload(ci, x):
        off = base + ci * _CH
        pltpu.async_copy(dst_e.at[pl.ds(off, _CH)], dbuf[x], lsem)
        pltpu.async_copy(ew.at[pl.ds(off, _CH)], wbuf[x], lsem)

    def _dwait(ci, x):
        off = base + ci * _CH
        pltpu.make_async_copy(dst_e.at[pl.ds(off, _CH)], dbuf[x], lsem).wait()
        pltpu.make_async_copy(ew.at[pl.ds(off, _CH)], wbuf[x], lsem).wait()

    def _fill_snap(ci, x, y):
        val = vals[y]

        def _f(k, _):
            w16 = wbuf[x][pl.ds(k * 16, 16)]
            for l in range(16):
                val[k * 16 + l, pl.ds(0, 16)] = jnp.full((16,), w16[l],
                                                         jnp.float32)
            return 0
        lax.fori_loop(0, _CH // 16, _f, 0)
        for k in range(_CH // 16):
            sidx[x][pl.ds(k * 16, 16)] = dbuf[x][pl.ds(k * 16, 16)]

    def _scatter(ci, x, y):
        pltpu.async_copy(vals[y], acc.at[sidx[x]], ssem[y], add=True)

    def _swait(ci, x, y):
        pltpu.make_async_copy(vals[y], acc.at[sidx[x]], ssem[y]).wait()

    def _step(ci, x, y):
        _dwait(ci, x)

        @pl.when(ci >= 2)
        def _():
            _swait(ci - 2, (x + 2) % 4, y)

        @pl.when(ci + 2 < _NCHUNK)
        def _():
            _dload(ci + 2, (x + 2) % 4)

        _fill_snap(ci, x, y)
        _scatter(ci, x, y)

    _dload(0, 0)
    _dload(1, 1)

    def _quad(k, _):
        for x in range(4):
            _step(4 * k + x, x, x % 2)
        return 0
    lax.fori_loop(0, _NCHUNK // 4, _quad, 0)
    _step(_NCHUNK - 1, (_NCHUNK - 1) % 4, (_NCHUNK - 1) % 2)
    _swait(_NCHUNK - 2, (_NCHUNK - 2) % 4, (_NCHUNK - 2) % 2)
    _swait(_NCHUNK - 1, (_NCHUNK - 1) % 4, (_NCHUNK - 1) % 2)

    plsc.subcore_barrier()
    _copy_out(acc, degpart, c, s)


def _deg_call(dst_e, ew):
    k = pl.kernel(
        _deg_body,
        out_type=jax.ShapeDtypeStruct((_NC, _N, _D), jnp.float32),
        mesh=_mesh,
        scratch_types=(
            [pltpu.VMEM((_CH,), jnp.int32)] * 4
            + [pltpu.VMEM((_CH,), jnp.float32)] * 4
            + [pltpu.VMEM((_CH,), jnp.int32)] * 4
            + [pltpu.VMEM((_CH, _D), jnp.float32)] * 2
            + [pltpu.VMEM_SHARED((_N, _D), jnp.float32)]
            + [pltpu.SemaphoreType.DMA] * 3
        ),
    )
    return k(dst_e, ew)


def _prop_body(nj, src_e, dst_e, ew, g, spart,
               a0, a1, a2, a3, b0, b1, b2, b3, e0, e1, e2, e3,
               q0, q1, q2, q3, r0, r1, r2, r3, acc,
               lsem, g0, g1, g2, g3, s0, s1, s2, s3):
    c = lax.axis_index("c")
    s = lax.axis_index("s")
    wid = c * _NS + s
    base = wid * _EPW

    sch = (a0, a1, a2, a3)
    dch = (b0, b1, b2, b3)
    ech = (e0, e1, e2, e3)
    sidx = (q0, q1, q2, q3)
    rows = (r0, r1, r2, r3)
    gsem = (g0, g1, g2, g3)
    ssem = (s0, s1, s2, s3)

    _zero_rows(r0)
    _zero_acc_slice(r0, acc, s, lsem)
    plsc.subcore_barrier()

    def _load(ci, x):
        off = base + ci * _CH
        pltpu.async_copy(src_e.at[pl.ds(off, _CH)], sch[x], lsem)
        pltpu.async_copy(dst_e.at[pl.ds(off, _CH)], dch[x], lsem)
        pltpu.async_copy(ew.at[pl.ds(off, _CH)], ech[x], lsem)

    def _lwait(ci, x):
        off = base + ci * _CH
        pltpu.make_async_copy(src_e.at[pl.ds(off, _CH)], sch[x], lsem).wait()
        pltpu.make_async_copy(dst_e.at[pl.ds(off, _CH)], dch[x], lsem).wait()
        pltpu.make_async_copy(ew.at[pl.ds(off, _CH)], ech[x], lsem).wait()

    def _gather(ci, x):
        pltpu.async_copy(g.at[sch[x]], rows[x], gsem[x])

    def _gwait(ci, x):
        pltpu.make_async_copy(g.at[sch[x]], rows[x], gsem[x]).wait()

    def _scatter(ci, x):
        pltpu.async_copy(rows[x], acc.at[sidx[x]], ssem[x], add=True)

    def _swait(ci, x):
        pltpu.make_async_copy(rows[x], acc.at[sidx[x]], ssem[x]).wait()

    def _scale_snap(ci, x):
        buf = rows[x]

        def _sc(k, _):
            w16 = ech[x][pl.ds(k * 16, 16)]
            for l in range(16):
                w = w16[l]
                i = k * 16 + l
                for j in range(nj):
                    buf[i, pl.ds(j * 16, 16)] = buf[i, pl.ds(j * 16, 16)] * w
            return 0
        lax.fori_loop(0, _CH // 16, _sc, 0)
        for k in range(_CH // 16):
            sidx[x][pl.ds(k * 16, 16)] = dch[x][pl.ds(k * 16, 16)]

    def _step(ci, x):
        @pl.when(ci + 3 < _NCHUNK)
        def _():
            _load(ci + 3, (x + 3) % 4)

        @pl.when(ci + 2 < _NCHUNK)
        def _():
            _lwait(ci + 2, (x + 2) % 4)

            @pl.when(ci >= 2)
            def _():
                _swait(ci - 2, (x + 2) % 4)
            _gather(ci + 2, (x + 2) % 4)

        _gwait(ci, x)
        _scale_snap(ci, x)
        _scatter(ci, x)

    _load(0, 0)
    _load(1, 1)
    _load(2, 2)
    _lwait(0, 0)
    _gather(0, 0)
    _lwait(1, 1)
    _gather(1, 1)

    def _quad(k, _):
        for x in range(4):
            _step(4 * k + x, x)
        return 0
    lax.fori_loop(0, _NCHUNK // 4, _quad, 0)
    _step(_NCHUNK - 1, (_NCHUNK - 1) % 4)
    for ci in range(_NCHUNK - 4, _NCHUNK):
        _swait(ci, ci % 4)

    plsc.subcore_barrier()
    _copy_out(acc, spart, c, s)


def _prop_call(src_e, dst_e, ew, g, d_valid):
    nj = d_valid // 16
    k = pl.kernel(
        functools.partial(_prop_body, nj),
        out_type=jax.ShapeDtypeStruct((_NC, _N, _D), jnp.float32),
        mesh=_mesh,
        scratch_types=(
            [pltpu.VMEM((_CH,), jnp.int32)] * 8
            + [pltpu.VMEM((_CH,), jnp.float32)] * 4
            + [pltpu.VMEM((_CH,), jnp.int32)] * 4
            + [pltpu.VMEM((_CH, _D), jnp.float32)] * 4
            + [pltpu.VMEM_SHARED((_N, _D), jnp.float32)]
            + [pltpu.SemaphoreType.DMA] * 9
        ),
    )
    return k(src_e, dst_e, ew, g)


_BLK = 1000


def _mm(a, w):
    return jnp.dot(a, w, preferred_element_type=jnp.float32,
                   precision=lax.Precision.HIGHEST)


def _tc_lin_body(x_r, w_r, out_r):
    out_r[...] = _mm(x_r[...], w_r[...])


def _tc_lin(x, W1p):
    return pl.pallas_call(
        _tc_lin_body,
        grid=(_N // _BLK,),
        in_specs=[
            pl.BlockSpec((_BLK, _D), lambda i: (i, 0)),
            pl.BlockSpec((_D, _D), lambda i: (0, 0)),
        ],
        out_specs=pl.BlockSpec((_BLK, _D), lambda i: (i, 0)),
        out_shape=jax.ShapeDtypeStruct((_N, _D), jnp.float32),
    )(x, W1p)


def _tc_gd_body(dp_r, lin_r, g_r, dv_r):
    deg = dp_r[0, :, 0] + dp_r[1, :, 0] + 1.0
    dinv = jnp.where(deg > 0, lax.rsqrt(deg), 0.0)
    g_r[...] = dinv[:, None] * lin_r[...]
    dv_r[...] = dinv[:, None]


def _tc_gd(degpart, lin):
    return pl.pallas_call(
        _tc_gd_body,
        grid=(_N // _BLK,),
        in_specs=[
            pl.BlockSpec((_NC, _BLK, _D), lambda i: (0, i, 0)),
            pl.BlockSpec((_BLK, _D), lambda i: (i, 0)),
        ],
        out_specs=(pl.BlockSpec((_BLK, _D), lambda i: (i, 0)),
                   pl.BlockSpec((_BLK, 1), lambda i: (i, 0))),
        out_shape=(jax.ShapeDtypeStruct((_N, _D), jnp.float32),
                   jax.ShapeDtypeStruct((_N, 1), jnp.float32)),
    )(degpart, lin)


def _tc_mid_body(final, s_r, g_r, dv_r, b_r, w_r, bd_r, out_r):
    dinv = dv_r[...][:, 0]
    stot = s_r[0] + s_r[1] + g_r[...]
    h = jax.nn.relu(dinv[:, None] * stot + b_r[...][None, :])
    z = _mm(h, w_r[...])
    if final:
        out_r[...] = z + bd_r[...][None, :]
    else:
        out_r[...] = dinv[:, None] * z


def _tc_mid(S, g, dinvcol, bp, Wp, bd, final):
    return pl.pallas_call(
        functools.partial(_tc_mid_body, final),
        grid=(_N // _BLK,),
        in_specs=[
            pl.BlockSpec((_NC, _BLK, _D), lambda i: (0, i, 0)),
            pl.BlockSpec((_BLK, _D), lambda i: (i, 0)),
            pl.BlockSpec((_BLK, 1), lambda i: (i, 0)),
            pl.BlockSpec((_D,), lambda i: (0,)),
            pl.BlockSpec((_D, _D), lambda i: (0, 0)),
            pl.BlockSpec((_D,), lambda i: (0,)),
        ],
        out_specs=pl.BlockSpec((_BLK, _D), lambda i: (i, 0)),
        out_shape=jax.ShapeDtypeStruct((_N, _D), jnp.float32),
    )(S, g, dinvcol, bp, Wp, bd)


def _pad2(w):
    return jnp.zeros((_D, _D), jnp.float32).at[:w.shape[0], :w.shape[1]].set(w)


def _pad1(b):
    return jnp.zeros((_D,), jnp.float32).at[:b.shape[0]].set(b)


def kernel(x, edgeIndex, edgeWeight, W1, b1, W2, b2, W3, b3, W4, b4, Wd, bd):
    src_e = edgeIndex[0]
    dst_e = edgeIndex[1]
    zb = jnp.zeros((_D,), jnp.float32)
    degpart = _deg_call(dst_e, edgeWeight)
    lin = _tc_lin(x, _pad2(W1))
    g, dinvcol = _tc_gd(degpart, lin)
    for d_in, b, W in ((16, b1, W2), (32, b2, W3), (64, b3, W4)):
        S = _prop_call(src_e, dst_e, edgeWeight, g, d_in)
        g = _tc_mid(S, g, dinvcol, _pad1(b), _pad2(W), zb, final=False)
    S = _prop_call(src_e, dst_e, edgeWeight, g, 128)
    return _tc_mid(S, g, dinvcol, _pad1(b4), Wd, bd, final=True)
